# baseline (speedup 1.0000x reference)
import functools

import jax
import jax.numpy as jnp
from jax import lax
from jax.experimental import pallas as pl
from jax.experimental.pallas import tpu as pltpu
from jax.experimental.pallas import tpu_sc as plsc

UNITS = 4
LB = -17.0
UB = 8.0
STEP = 0.05
NUM_BUCKETS = int((UB - LB) / STEP) + 1
RESIDUE = LB - STEP

_NB_PAD = 512
_U_PAD = 8
_B = 16384
_NW = 32
_ROWS = _B // _NW
_VECS = _ROWS * UNITS // 16


def _prep_body(v_ref, w_ref, q_ref):
    v = v_ref[...]
    w = jnp.maximum(v, 0.0)
    row = lax.broadcasted_iota(jnp.int32, (_NB_PAD, _NB_PAD), 0)
    col = lax.broadcasted_iota(jnp.int32, (_NB_PAD, _NB_PAD), 1)
    m = jnp.where(row < col, jnp.float32(1.0), jnp.float32(0.0))
    p = jax.lax.dot(w, m, precision=jax.lax.Precision.HIGHEST)
    w_ref[...] = w
    q_ref[...] = p * jnp.float32(STEP)


def _prep_tables(v_pad):
    return pl.pallas_call(
        _prep_body,
        out_shape=[
            jax.ShapeDtypeStruct((_U_PAD, _NB_PAD), jnp.float32),
            jax.ShapeDtypeStruct((_U_PAD, _NB_PAD), jnp.float32),
        ],
    )(v_pad)


def _sc_body(x_hbm, q_hbm, w_hbm, b_hbm, out_hbm, x_v, q_v, w_v, b_v, o_v):
    wid = lax.axis_index("s") * 2 + lax.axis_index("c")
    base = wid * _ROWS
    pltpu.sync_copy(x_hbm.at[pl.ds(base, _ROWS), :], x_v)
    pltpu.sync_copy(q_hbm, q_v)
    pltpu.sync_copy(w_hbm, w_v)
    pltpu.sync_copy(b_hbm, b_v)

    lane = lax.iota(jnp.int32, 16)
    u_vec = lax.bitwise_and(lane, 3)
    r_vec = lax.shift_right_logical(lane, 2)
    bias = plsc.load_gather(b_v, [u_vec]) + jnp.float32(RESIDUE)

    c_lb = jnp.float32(LB + 1e-09)
    c_ub = jnp.float32(UB - 1e-09)
    c_lbs = jnp.float32(LB)
    c_step = jnp.float32(STEP)

    def body(i, _):
        rows = r_vec + i * 4
        xv = plsc.load_gather(x_v, [rows, u_vec])
        xc = jnp.minimum(jnp.maximum(xv, c_lb), c_ub)
        t = (xc - c_lbs + c_step) / c_step
        k = t.astype(jnp.int32)
        k = jnp.minimum(jnp.maximum(k, 0), NUM_BUCKETS - 1)
        delta = xc - c_lbs + c_step - k.astype(jnp.float32) * c_step
        qv = plsc.load_gather(q_v, [u_vec, k])
        wv = plsc.load_gather(w_v, [u_vec, k])
        z = qv + delta * wv + bias
        s = jnp.float32(1.0) / (jnp.float32(1.0) + jnp.exp(-z))
        plsc.store_scatter(o_v, [rows, u_vec], s)
        return 0

    lax.fori_loop(0, _VECS, body, 0)
    pltpu.sync_copy(o_v, out_hbm.at[pl.ds(base, _ROWS), :])


@jax.jit
def _sc_main(x, q_tab, w_tab, b_pad):
    mesh = plsc.VectorSubcoreMesh(core_axis_name="c", subcore_axis_name="s")
    f = pl.kernel(
        _sc_body,
        mesh=mesh,
        compiler_params=pltpu.CompilerParams(
            needs_layout_passes=False, use_tc_tiling_on_sc=False
        ),
        out_type=jax.ShapeDtypeStruct((_B, UNITS), jnp.float32),
        scratch_types=[
            pltpu.VMEM((_ROWS, UNITS), jnp.float32),
            pltpu.VMEM((_U_PAD, _NB_PAD), jnp.float32),
            pltpu.VMEM((_U_PAD, _NB_PAD), jnp.float32),
            pltpu.VMEM((16,), jnp.float32),
            pltpu.VMEM((_ROWS, UNITS), jnp.float32),
        ],
    )
    return f(x, q_tab, w_tab, b_pad)


def kernel(x, v, b):
    if x.ndim == 1:
        x = jnp.broadcast_to(x[:, None], (x.shape[0], UNITS))
    v_pad = jnp.zeros((_U_PAD, _NB_PAD), jnp.float32).at[:UNITS, :NUM_BUCKETS].set(v)
    w_tab, q_tab = _prep_tables(v_pad)
    b_pad = jnp.zeros((16,), jnp.float32).at[:UNITS].set(b)
    return _sc_main(x, q_tab, w_tab, b_pad)


# speedup vs baseline: 1.2875x; 1.2875x over previous
import functools

import jax
import jax.numpy as jnp
from jax import lax
from jax.experimental import pallas as pl
from jax.experimental.pallas import tpu as pltpu
from jax.experimental.pallas import tpu_sc as plsc

UNITS = 4
LB = -17.0
UB = 8.0
STEP = 0.05
NUM_BUCKETS = int((UB - LB) / STEP) + 1
RESIDUE = LB - STEP

_NB_PAD = 512
_U_PAD = 8
_B = 16384
_NW = 32
_ROWS = _B // _NW
_VECS = _ROWS * UNITS // 16


def _prep_body(v_ref, w_ref, q_ref):
    v = v_ref[...]
    w = jnp.maximum(v, 0.0)
    row = lax.broadcasted_iota(jnp.int32, (_NB_PAD, _NB_PAD), 0)
    col = lax.broadcasted_iota(jnp.int32, (_NB_PAD, _NB_PAD), 1)
    m = jnp.where(row < col, jnp.float32(1.0), jnp.float32(0.0))
    p = jax.lax.dot(w, m, precision=jax.lax.Precision.HIGHEST)
    w_ref[...] = w
    q_ref[...] = p * jnp.float32(STEP)


def _prep_tables(v_pad):
    return pl.pallas_call(
        _prep_body,
        out_shape=[
            jax.ShapeDtypeStruct((_U_PAD, _NB_PAD), jnp.float32),
            jax.ShapeDtypeStruct((_U_PAD, _NB_PAD), jnp.float32),
        ],
    )(v_pad)


_CROWS = 256
_NCHUNK = _ROWS // _CROWS
_CVECS = _CROWS * UNITS // 16


def _sc_body(x_hbm, q_hbm, w_hbm, b_hbm, out_hbm, x_v, q_v, w_v, b_v, o_v):
    wid = lax.axis_index("s") * 2 + lax.axis_index("c")
    base = wid * _ROWS
    pltpu.sync_copy(q_hbm, q_v)
    pltpu.sync_copy(w_hbm, w_v)
    pltpu.sync_copy(b_hbm, b_v)

    lane = lax.iota(jnp.int32, 16)
    u_vec = lax.bitwise_and(lane, 3)
    r_vec = lax.shift_right_logical(lane, 2)
    bias = plsc.load_gather(b_v, [u_vec]) + jnp.float32(RESIDUE)

    c_lb = jnp.float32(LB + 1e-09)
    c_ub = jnp.float32(UB - 1e-09)
    c_lbs = jnp.float32(LB)
    c_step = jnp.float32(STEP)

    def chunk(c, _):
        crow = base + c * _CROWS
        pltpu.sync_copy(x_hbm.at[pl.ds(crow, _CROWS), :], x_v)

        def body(i, _):
            rows = r_vec + i * 4
            xv = plsc.load_gather(x_v, [rows, u_vec])
            xc = jnp.minimum(jnp.maximum(xv, c_lb), c_ub)
            t = (xc - c_lbs + c_step) / c_step
            k = t.astype(jnp.int32)
            k = jnp.minimum(jnp.maximum(k, 0), NUM_BUCKETS - 1)
            delta = xc - c_lbs + c_step - k.astype(jnp.float32) * c_step
            qv = plsc.load_gather(q_v, [u_vec, k])
            wv = plsc.load_gather(w_v, [u_vec, k])
            z = qv + delta * wv + bias
            s = jnp.float32(1.0) / (jnp.float32(1.0) + jnp.exp(-z))
            plsc.store_scatter(o_v, [rows, u_vec], s)
            return 0

        lax.fori_loop(0, _CVECS, body, 0)
        pltpu.sync_copy(o_v, out_hbm.at[pl.ds(crow, _CROWS), :])
        return 0

    lax.fori_loop(0, _NCHUNK, chunk, 0)


@jax.jit
def _sc_main(x, q_tab, w_tab, b_pad):
    mesh = plsc.VectorSubcoreMesh(core_axis_name="c", subcore_axis_name="s")
    f = pl.kernel(
        _sc_body,
        mesh=mesh,
        compiler_params=pltpu.CompilerParams(needs_layout_passes=False),
        out_type=jax.ShapeDtypeStruct((_B, UNITS), jnp.float32),
        scratch_types=[
            pltpu.VMEM((_CROWS, UNITS), jnp.float32),
            pltpu.VMEM((_U_PAD, _NB_PAD), jnp.float32),
            pltpu.VMEM((_U_PAD, _NB_PAD), jnp.float32),
            pltpu.VMEM((16,), jnp.float32),
            pltpu.VMEM((_CROWS, UNITS), jnp.float32),
        ],
    )
    return f(x, q_tab, w_tab, b_pad)


def kernel(x, v, b):
    if x.ndim == 1:
        x = jnp.broadcast_to(x[:, None], (x.shape[0], UNITS))
    v_pad = jnp.zeros((_U_PAD, _NB_PAD), jnp.float32).at[:UNITS, :NUM_BUCKETS].set(v)
    w_tab, q_tab = _prep_tables(v_pad)
    b_pad = jnp.zeros((16,), jnp.float32).at[:UNITS].set(b)
    return _sc_main(x, q_tab, w_tab, b_pad)
